# Initial kernel scaffold; baseline (speedup 1.0000x reference)
import jax
import jax.numpy as jnp
from jax.experimental import pallas as pl


def kernel(x_bd):
    raise NotImplementedError("write your pallas kernel here")


# speedup vs baseline: 2.7134x; 2.7134x over previous
import functools

import jax
import jax.numpy as jnp
from jax import lax
from jax.experimental import pallas as pl
from jax.experimental.pallas import tpu as pltpu
from jax.experimental.pallas import tpu_sc as plsc

_MARK_AX = 0
_OP_SHL = 1
_OP_SHR = 2
_ALU_LO = 3
_ALU_HI = 19
_AX_CARRY_LO = 35
_OUTPUT_LO = 76
_OUTPUT_HI = 92
_D = 108

_NUM_CORES = 2
_NUM_SUBCORES = 16
_NW = _NUM_CORES * _NUM_SUBCORES


def _first_hot_vec(window):
    ffs = plsc.all_reduce_ffs(window > 0.5)
    return jnp.where(ffs >= 16, 0, ffs).astype(jnp.int32)


def _make_kernel(rows, rb, interpret=False):
    rows_per_w = rows // _NW
    nblk = rows_per_w // rb
    mesh = plsc.VectorSubcoreMesh(
        core_axis_name="c",
        subcore_axis_name="s",
        num_cores=_NUM_CORES,
        num_subcores=_NUM_SUBCORES,
    )

    @functools.partial(
        pl.kernel,
        out_type=jax.ShapeDtypeStruct((rows * _D,), jnp.float32),
        mesh=mesh,
        scratch_types=[pltpu.VMEM((rb * _D,), jnp.float32)],
        compiler_params=pltpu.CompilerParams(needs_layout_passes=False),
        interpret=interpret,
    )
    def k(x_hbm, out_hbm, buf):
        wid = lax.axis_index("s") * _NUM_CORES + lax.axis_index("c")
        base = wid * (rows_per_w * _D)
        lanes = lax.iota(jnp.int32, 16)

        def row_body(r, carry):
            rb_off = r * _D
            v_lo = buf[pl.ds(rb_off + _ALU_LO, 16)]
            v_hi = buf[pl.ds(rb_off + _ALU_HI, 16)]
            v_s = buf[pl.ds(rb_off + _AX_CARRY_LO, 16)]
            a_lo = _first_hot_vec(v_lo)
            a_hi = _first_hot_vec(v_hi)
            s = _first_hot_vec(v_s)
            val = a_hi * 16 + a_lo
            shl = lax.shift_left(val, s) & 255
            shr = lax.shift_right_logical(val, s)
            v_hdr = buf[pl.ds(rb_off, 16)]
            mark = v_hdr[_MARK_AX]
            f_shl = v_hdr[_OP_SHL]
            f_shr = v_hdr[_OP_SHR]
            op_shl = f_shl > 0.5
            op_shr = jnp.logical_and(jnp.logical_not(op_shl), f_shr > 0.5)
            active = jnp.logical_and(
                mark >= 0.5, jnp.logical_or(op_shl, op_shr)
            )
            res = jnp.where(op_shl, shl, shr)
            r_lo = res & 15
            r_hi = lax.shift_right_logical(res, 4)
            add = jnp.where(active, jnp.float32(2.0), jnp.float32(0.0))
            w1 = buf[pl.ds(rb_off + _OUTPUT_LO, 16)]
            buf[pl.ds(rb_off + _OUTPUT_LO, 16)] = w1 + jnp.where(
                lanes == r_lo, add, jnp.float32(0.0)
            )
            w2 = buf[pl.ds(rb_off + _OUTPUT_HI, 16)]
            buf[pl.ds(rb_off + _OUTPUT_HI, 16)] = w2 + jnp.where(
                lanes == r_hi, add, jnp.float32(0.0)
            )
            return carry

        def blk_body(b, carry):
            off = base + b * (rb * _D)
            pltpu.sync_copy(x_hbm.at[pl.ds(off, rb * _D)], buf)
            lax.fori_loop(0, rb, row_body, 0, unroll=False)
            pltpu.sync_copy(buf, out_hbm.at[pl.ds(off, rb * _D)])
            return carry

        lax.fori_loop(0, nblk, blk_body, 0, unroll=False)

    return k


def kernel(x_bd, interpret=False, rb=256):
    b, seq, d = x_bd.shape
    assert d == _D
    rows = b * seq
    rb = min(rb, rows // _NW)
    assert rows % (_NW * rb) == 0
    flat = x_bd.reshape(rows * d)
    out = _make_kernel(rows, rb, interpret=interpret)(flat)
    return out.reshape(b, seq, d)


# speedup vs baseline: 11.3173x; 4.1708x over previous
import functools

import jax
import jax.numpy as jnp
from jax import lax
from jax.experimental import pallas as pl
from jax.experimental.pallas import tpu as pltpu
from jax.experimental.pallas import tpu_sc as plsc

_MARK_AX = 0
_OP_SHL = 1
_OP_SHR = 2
_ALU_LO = 3
_ALU_HI = 19
_AX_CARRY_LO = 35
_OUTPUT_LO = 76
_OUTPUT_HI = 92
_D = 108

_B = 16
_SEQ = 8192
_ROWS = _D * _B
_NUM_CORES = 2
_NUM_SUBCORES = 16
_NW = _NUM_CORES * _NUM_SUBCORES

_SPAN = 128
_NCHUNK = 2 * (_SEQ // _SPAN)
_PER_W = _NCHUNK // _NW


def _make_kernel():
    mesh = plsc.VectorSubcoreMesh(
        core_axis_name="c",
        subcore_axis_name="s",
        num_cores=_NUM_CORES,
        num_subcores=_NUM_SUBCORES,
    )

    @functools.partial(
        pl.kernel,
        out_type=jax.ShapeDtypeStruct((_ROWS, _SEQ), jnp.float32),
        mesh=mesh,
        scratch_types=[
            pltpu.VMEM((_D, 8, _SPAN), jnp.float32),
            pltpu.VMEM((8, _SPAN), jnp.int32),
            pltpu.VMEM((8, _SPAN), jnp.int32),
            pltpu.SemaphoreType.DMA,
        ],
        compiler_params=pltpu.CompilerParams(
            needs_layout_passes=False, use_tc_tiling_on_sc=True
        ),
    )
    def k(x_hbm, out_hbm, big, rlo_buf, rhi_buf, sem):
        wid = lax.axis_index("s") * _NUM_CORES + lax.axis_index("c")

        def run_chunk(c):
            g = c % 2
            s0 = (c // 2) * _SPAN

            def in_issue(d, carry):
                pltpu.async_copy(
                    x_hbm.at[pl.ds(d * _B + g * 8, 8), pl.ds(s0, _SPAN)],
                    big.at[d],
                    sem,
                )
                return carry

            def in_drain(d, carry):
                pltpu.make_async_copy(
                    x_hbm.at[pl.ds(d * _B + g * 8, 8), pl.ds(s0, _SPAN)],
                    big.at[d],
                    sem,
                ).wait()
                return carry

            lax.fori_loop(0, _D, in_issue, 0, unroll=False)
            lax.fori_loop(0, _D, in_drain, 0, unroll=False)

            def slot_body(slot, carry):
                r = slot // 8
                c16 = (slot % 8) * 16
                sixteen = jnp.full((16,), 16, jnp.int32)
                a_lo = sixteen
                a_hi = sixteen
                sh = sixteen
                for kk in range(16):
                    v = big[_ALU_LO + kk, r, pl.ds(c16, 16)]
                    a_lo = jnp.minimum(
                        a_lo, jnp.where(v > 0.5, jnp.int32(kk), sixteen)
                    )
                for kk in range(16):
                    v = big[_ALU_HI + kk, r, pl.ds(c16, 16)]
                    a_hi = jnp.minimum(
                        a_hi, jnp.where(v > 0.5, jnp.int32(kk), sixteen)
                    )
                for kk in range(16):
                    v = big[_AX_CARRY_LO + kk, r, pl.ds(c16, 16)]
                    sh = jnp.minimum(
                        sh, jnp.where(v > 0.5, jnp.int32(kk), sixteen)
                    )
                zero = jnp.zeros((16,), jnp.int32)
                a_lo = jnp.where(a_lo >= 16, zero, a_lo)
                a_hi = jnp.where(a_hi >= 16, zero, a_hi)
                sh = jnp.where(sh >= 16, zero, sh)
                val = a_hi * 16 + a_lo
                shl = lax.shift_left(val, sh) & 255
                shr = lax.shift_right_logical(val, sh)
                mark = big[_MARK_AX, r, pl.ds(c16, 16)] >= 0.5
                f1 = big[_OP_SHL, r, pl.ds(c16, 16)] > 0.5
                f2 = big[_OP_SHR, r, pl.ds(c16, 16)] > 0.5
                active = jnp.logical_and(mark, jnp.logical_or(f1, f2))
                res = jnp.where(f1, shl, shr)
                r_lo = res & 15
                r_hi = lax.shift_right_logical(res, 4)
                rlo_buf[r, pl.ds(c16, 16)] = jnp.where(active, r_lo, sixteen)
                rhi_buf[r, pl.ds(c16, 16)] = jnp.where(active, r_hi, sixteen)
                return carry

            lax.fori_loop(0, 64, slot_body, 0, unroll=False)

            def upd_body(slot, carry):
                r = slot // 8
                c16 = (slot % 8) * 16
                r_lo = rlo_buf[r, pl.ds(c16, 16)]
                r_hi = rhi_buf[r, pl.ds(c16, 16)]
                two = jnp.full((16,), 2.0, jnp.float32)
                zf = jnp.zeros((16,), jnp.float32)
                for j in range(16):
                    d = _OUTPUT_LO + j
                    v = big[d, r, pl.ds(c16, 16)]
                    big[d, r, pl.ds(c16, 16)] = v + jnp.where(
                        r_lo == j, two, zf
                    )
                for j in range(16):
                    d = _OUTPUT_HI + j
                    v = big[d, r, pl.ds(c16, 16)]
                    big[d, r, pl.ds(c16, 16)] = v + jnp.where(
                        r_hi == j, two, zf
                    )
                return carry

            lax.fori_loop(0, 64, upd_body, 0, unroll=False)

            def out_issue(d, carry):
                pltpu.async_copy(
                    big.at[d],
                    out_hbm.at[pl.ds(d * _B + g * 8, 8), pl.ds(s0, _SPAN)],
                    sem,
                )
                return carry

            def out_drain(d, carry):
                pltpu.make_async_copy(
                    big.at[d],
                    out_hbm.at[pl.ds(d * _B + g * 8, 8), pl.ds(s0, _SPAN)],
                    sem,
                ).wait()
                return carry

            lax.fori_loop(0, _D, out_issue, 0, unroll=False)
            lax.fori_loop(0, _D, out_drain, 0, unroll=False)

        def chunk_body(i, carry):
            run_chunk(wid * _PER_W + i)
            return carry

        lax.fori_loop(0, _PER_W, chunk_body, 0, unroll=False)

    return k


def kernel(x_bd):
    b, seq, d = x_bd.shape
    assert (b, seq, d) == (_B, _SEQ, _D)
    x2 = x_bd.transpose(2, 0, 1).reshape(_ROWS, _SEQ)
    out = _make_kernel()(x2)
    return out.reshape(_D, _B, _SEQ).transpose(1, 2, 0)
